# Initial kernel scaffold; baseline (speedup 1.0000x reference)
import functools

import jax
import jax.numpy as jnp
from jax import lax
from jax.experimental import pallas as pl
from jax.experimental.pallas import tpu as pltpu
from jax.experimental.pallas import tpu_sc as plsc

IN_DIM = 4096
HID = 4096
FS = IN_DIM * HID
NSUB = 16
NCORE = 2
SLICE = 2080768
NSLICE = -(-FS // SLICE)
SLICES_SC0 = 5
PART = SLICE // NSUB
CHUNK = 4096
JROWS = CHUNK // 128


def _scatter_w(rows_p, cols_p, vals_p, zeros):
    nnzp = rows_p.shape[0]
    share = nnzp // NSUB
    n_chunks = share // CHUNK

    mesh = plsc.VectorSubcoreMesh(core_axis_name="c", subcore_axis_name="s")

    @functools.partial(
        pl.kernel,
        out_type=jax.ShapeDtypeStruct((FS,), jnp.float32),
        mesh=mesh,
        scratch_types=[
            pltpu.VMEM((CHUNK,), jnp.int32),
            pltpu.VMEM((CHUNK,), jnp.int32),
            pltpu.VMEM((CHUNK,), jnp.float32),
            pltpu.VMEM((JROWS, 128), jnp.int32),
            pltpu.VMEM((JROWS, 128), jnp.float32),
            pltpu.VMEM_SHARED((SLICE,), jnp.float32),
            pltpu.SemaphoreType.DMA,
        ],
    )
    def scatter(rows_hbm, cols_hbm, vals_hbm, zeros_hbm, w_hbm,
                r_v, c_v, v_v, idx2, val2, acc, sem):
        c = lax.axis_index("c")
        s = lax.axis_index("s")
        my_off = s * share
        lo = c * SLICES_SC0
        hi = lax.select(c == 0, SLICES_SC0, NSLICE)

        @pl.loop(0, NSLICE)
        def _slice(k):
            @pl.when(jnp.logical_and(k >= lo, k < hi))
            def _():
                base = k * SLICE
                span = jnp.minimum(SLICE, FS - base)
                my_zero = jnp.minimum(jnp.maximum(span - s * PART, 0), PART)

                @pl.when(my_zero > 0)
                def _():
                    pltpu.sync_copy(zeros_hbm.at[pl.ds(s * PART, my_zero)],
                                    acc.at[pl.ds(s * PART, my_zero)])
                plsc.subcore_barrier()

                @pl.loop(0, n_chunks)
                def _chunk(i):
                    off = my_off + i * CHUNK
                    d1 = pltpu.async_copy(
                        rows_hbm.at[pl.ds(off, CHUNK)], r_v, sem)
                    d2 = pltpu.async_copy(
                        cols_hbm.at[pl.ds(off, CHUNK)], c_v, sem)
                    d3 = pltpu.async_copy(
                        vals_hbm.at[pl.ds(off, CHUNK)], v_v, sem)
                    d1.wait()
                    d2.wait()
                    d3.wait()

                    @pl.loop(0, JROWS)
                    def _row(j):
                        for l in range(8):
                            g = j * 128 + l * 16
                            rv = r_v[pl.ds(g, 16)]
                            cv = c_v[pl.ds(g, 16)]
                            vv = v_v[pl.ds(g, 16)]
                            flat = rv * HID + cv
                            inb = jnp.logical_and(flat >= base,
                                                  flat < base + SLICE)
                            idx2[j, pl.ds(l * 16, 16)] = jnp.where(
                                inb, flat - base, 0)
                            val2[j, pl.ds(l * 16, 16)] = jnp.where(
                                inb, vv, 0.0)
                        pltpu.sync_copy(val2.at[j], acc.at[idx2.at[j]],
                                        add=True)

                plsc.subcore_barrier()

                @pl.when(my_zero > 0)
                def _():
                    pltpu.sync_copy(
                        acc.at[pl.ds(s * PART, my_zero)],
                        w_hbm.at[pl.ds(base + s * PART, my_zero)])
                plsc.subcore_barrier()

    return scatter(rows_p, cols_p, vals_p, zeros)


def _mm_body(x_ref, w_ref, b_ref, o_ref):
    xb = x_ref[...]
    wb = w_ref[...].astype(jnp.bfloat16)
    acc = jnp.dot(xb, wb, preferred_element_type=jnp.float32)
    o_ref[...] = jnp.tanh(acc + b_ref[...])


def _matmul(xb, w, bias2d):
    batch = xb.shape[0]
    bn = 512
    return pl.pallas_call(
        _mm_body,
        grid=(HID // bn,),
        in_specs=[
            pl.BlockSpec((batch, IN_DIM), lambda j: (0, 0)),
            pl.BlockSpec((IN_DIM, bn), lambda j: (0, j)),
            pl.BlockSpec((1, bn), lambda j: (0, j)),
        ],
        out_specs=pl.BlockSpec((batch, bn), lambda j: (0, j)),
        out_shape=jax.ShapeDtypeStruct((batch, HID), jnp.float32),
    )(xb, w, bias2d)


def kernel(x, values, bias, rows, cols):
    nnz = rows.shape[0]
    grain = NSUB * CHUNK
    nnzp = -(-nnz // grain) * grain
    pad = nnzp - nnz
    rows_p = jnp.concatenate(
        [rows, jnp.full((pad,), IN_DIM, jnp.int32)])
    cols_p = jnp.concatenate([cols, jnp.zeros((pad,), jnp.int32)])
    vals_p = jnp.concatenate([values, jnp.zeros((pad,), jnp.float32)])
    zeros = jnp.zeros((SLICE,), jnp.float32)

    w_flat = _scatter_w(rows_p, cols_p, vals_p, zeros)
    w = w_flat.reshape(IN_DIM, HID)
    xb = x.astype(jnp.bfloat16)
    return _matmul(xb, w, bias.reshape(1, HID))


# speedup vs baseline: 1.4332x; 1.4332x over previous
import functools

import jax
import jax.numpy as jnp
from jax import lax
from jax.experimental import pallas as pl
from jax.experimental.pallas import tpu as pltpu
from jax.experimental.pallas import tpu_sc as plsc

IN_DIM = 4096
HID = 4096
FS = IN_DIM * HID
NSUB = 16
NCORE = 2
SLICE = 1703936
NSLICE = -(-FS // SLICE)
PART = SLICE // NSUB
CHUNK = 4096
JROWS = CHUNK // 128


def _scatter_w(rows_p, cols_p, vals_p, zeros):
    nnzp = rows_p.shape[0]
    share = nnzp // NSUB
    n_chunks = share // CHUNK

    mesh = plsc.VectorSubcoreMesh(core_axis_name="c", subcore_axis_name="s")

    @functools.partial(
        pl.kernel,
        out_type=jax.ShapeDtypeStruct((NSLICE * SLICE,), jnp.float32),
        mesh=mesh,
        scratch_types=[
            pltpu.VMEM((CHUNK,), jnp.int32),
            pltpu.VMEM((CHUNK,), jnp.int32),
            pltpu.VMEM((CHUNK,), jnp.float32),
            pltpu.VMEM((JROWS, 128), jnp.int32),
            pltpu.VMEM((JROWS, 128), jnp.float32),
            pltpu.VMEM_SHARED((SLICE,), jnp.float32),
            pltpu.SemaphoreType.DMA,
        ],
    )
    def scatter(rows_hbm, cols_hbm, vals_hbm, zeros_hbm, w_hbm,
                r_v, c_v, v_v, idx2, val2, acc, sem):
        c = lax.axis_index("c")
        s = lax.axis_index("s")
        my_off = s * share

        @pl.loop(0, NSLICE)
        def _slice(k):
            @pl.when(k % 2 == c)
            def _():
                base = k * SLICE
                pltpu.sync_copy(zeros_hbm.at[pl.ds(s * PART, PART)],
                                acc.at[pl.ds(s * PART, PART)])
                plsc.subcore_barrier()

                @pl.loop(0, n_chunks)
                def _chunk(i):
                    off = my_off + i * CHUNK
                    d1 = pltpu.async_copy(
                        rows_hbm.at[pl.ds(off, CHUNK)], r_v, sem)
                    d2 = pltpu.async_copy(
                        cols_hbm.at[pl.ds(off, CHUNK)], c_v, sem)
                    d3 = pltpu.async_copy(
                        vals_hbm.at[pl.ds(off, CHUNK)], v_v, sem)
                    d1.wait()
                    d2.wait()
                    d3.wait()

                    @pl.loop(0, JROWS)
                    def _row(j):
                        for l in range(8):
                            g = j * 128 + l * 16
                            rv = r_v[pl.ds(g, 16)]
                            cv = c_v[pl.ds(g, 16)]
                            vv = v_v[pl.ds(g, 16)]
                            flat = rv * HID + cv
                            inb = jnp.logical_and(flat >= base,
                                                  flat < base + SLICE)
                            idx2[j, pl.ds(l * 16, 16)] = jnp.where(
                                inb, flat - base, 0)
                            val2[j, pl.ds(l * 16, 16)] = jnp.where(
                                inb, vv, 0.0)
                        pltpu.sync_copy(val2.at[j], acc.at[idx2.at[j]],
                                        add=True)

                plsc.subcore_barrier()
                pltpu.sync_copy(
                    acc.at[pl.ds(s * PART, PART)],
                    w_hbm.at[pl.ds(base + s * PART, PART)])
                plsc.subcore_barrier()

    return scatter(rows_p, cols_p, vals_p, zeros)


def _mm_body(x_ref, w_ref, b_ref, o_ref):
    xb = x_ref[...]
    wb = w_ref[...].astype(jnp.bfloat16)
    acc = jnp.dot(xb, wb, preferred_element_type=jnp.float32)
    o_ref[...] = jnp.tanh(acc + b_ref[...])


def _matmul(xb, w, bias2d):
    batch = xb.shape[0]
    bn = 512
    return pl.pallas_call(
        _mm_body,
        grid=(HID // bn,),
        in_specs=[
            pl.BlockSpec((batch, IN_DIM), lambda j: (0, 0)),
            pl.BlockSpec((IN_DIM, bn), lambda j: (0, j)),
            pl.BlockSpec((1, bn), lambda j: (0, j)),
        ],
        out_specs=pl.BlockSpec((batch, bn), lambda j: (0, j)),
        out_shape=jax.ShapeDtypeStruct((batch, HID), jnp.float32),
    )(xb, w, bias2d)


def kernel(x, values, bias, rows, cols):
    nnz = rows.shape[0]
    grain = NSUB * CHUNK
    nnzp = -(-nnz // grain) * grain
    pad = nnzp - nnz
    rows_p = jnp.concatenate(
        [rows, jnp.full((pad,), IN_DIM, jnp.int32)])
    cols_p = jnp.concatenate([cols, jnp.zeros((pad,), jnp.int32)])
    vals_p = jnp.concatenate([values, jnp.zeros((pad,), jnp.float32)])
    zeros = jnp.zeros((SLICE,), jnp.float32)

    w_flat = _scatter_w(rows_p, cols_p, vals_p, zeros)
    w = w_flat[:FS].reshape(IN_DIM, HID)
    xb = x.astype(jnp.bfloat16)
    return _matmul(xb, w, bias.reshape(1, HID))


# speedup vs baseline: 1.4347x; 1.0010x over previous
import functools

import jax
import jax.numpy as jnp
from jax import lax
from jax.experimental import pallas as pl
from jax.experimental.pallas import tpu as pltpu
from jax.experimental.pallas import tpu_sc as plsc

IN_DIM = 4096
HID = 4096
FS = IN_DIM * HID
NSUB = 16
NCORE = 2
SLICE = 1703936
NSLICE = -(-FS // SLICE)
PART = SLICE // NSUB
CHUNK = 2048
JROWS = CHUNK // 128
CBYTES = CHUNK * 4


def _scatter_w(rows_p, cols_p, vals_p, zeros):
    nnzp = rows_p.shape[0]
    share = nnzp // NSUB
    n_pairs = share // (2 * CHUNK)

    mesh = plsc.VectorSubcoreMesh(core_axis_name="c", subcore_axis_name="s")

    @functools.partial(
        pl.kernel,
        out_type=jax.ShapeDtypeStruct((NSLICE * SLICE,), jnp.float32),
        mesh=mesh,
        scratch_types=[
            pltpu.VMEM((CHUNK,), jnp.int32),
            pltpu.VMEM((CHUNK,), jnp.int32),
            pltpu.VMEM((CHUNK,), jnp.float32),
            pltpu.VMEM((CHUNK,), jnp.int32),
            pltpu.VMEM((CHUNK,), jnp.int32),
            pltpu.VMEM((CHUNK,), jnp.float32),
            pltpu.VMEM((JROWS, 128), jnp.int32),
            pltpu.VMEM((JROWS, 128), jnp.float32),
            pltpu.VMEM((JROWS, 128), jnp.int32),
            pltpu.VMEM((JROWS, 128), jnp.float32),
            pltpu.VMEM_SHARED((SLICE,), jnp.float32),
            pltpu.SemaphoreType.DMA,
            pltpu.SemaphoreType.DMA,
            pltpu.SemaphoreType.DMA,
        ],
    )
    def scatter(rows_hbm, cols_hbm, vals_hbm, zeros_hbm, w_hbm,
                r_a, c_a, v_a, r_b, c_b, v_b,
                idx_a, val_a, idx_b, val_b, acc,
                sem_a, sem_b, sem_st):
        c = lax.axis_index("c")
        s = lax.axis_index("s")
        my_off = s * share

        def load_chunk(ci, r_v, c_v, v_v, sem):
            off = my_off + ci * CHUNK
            pltpu.async_copy(rows_hbm.at[pl.ds(off, CHUNK)], r_v, sem)
            pltpu.async_copy(cols_hbm.at[pl.ds(off, CHUNK)], c_v, sem)
            pltpu.async_copy(vals_hbm.at[pl.ds(off, CHUNK)], v_v, sem)

        def wait_inputs(r_v, sem):
            for _ in range(3):
                pltpu.make_async_copy(
                    rows_hbm.at[pl.ds(0, CHUNK)], r_v, sem).wait()

        def drain_streams(v_v):
            pltpu.make_async_copy(
                zeros_hbm.at[pl.ds(0, CHUNK)], v_v, sem_st).wait()

        def compute_fire(base, r_v, c_v, v_v, idx2, val2):
            @pl.loop(0, JROWS)
            def _row(j):
                for l in range(8):
                    g = j * 128 + l * 16
                    rv = r_v[pl.ds(g, 16)]
                    cv = c_v[pl.ds(g, 16)]
                    vv = v_v[pl.ds(g, 16)]
                    flat = rv * HID + cv
                    inb = jnp.logical_and(flat >= base,
                                          flat < base + SLICE)
                    idx2[j, pl.ds(l * 16, 16)] = jnp.where(
                        inb, flat - base, 0)
                    val2[j, pl.ds(l * 16, 16)] = jnp.where(inb, vv, 0.0)
                pltpu.async_copy(val2.at[j], acc.at[idx2.at[j]], sem_st,
                                 add=True)

        @pl.loop(0, NSLICE)
        def _slice(k):
            @pl.when(k % 2 == c)
            def _():
                base = k * SLICE
                pltpu.sync_copy(zeros_hbm.at[pl.ds(s * PART, PART)],
                                acc.at[pl.ds(s * PART, PART)])
                plsc.subcore_barrier()

                load_chunk(0, r_a, c_a, v_a, sem_a)
                load_chunk(1, r_b, c_b, v_b, sem_b)

                @pl.loop(0, n_pairs)
                def _pair(i):
                    @pl.when(i > 0)
                    def _():
                        drain_streams(v_a)
                        drain_streams(v_b)

                    wait_inputs(r_a, sem_a)
                    compute_fire(base, r_a, c_a, v_a, idx_a, val_a)

                    @pl.when(i < n_pairs - 1)
                    def _():
                        load_chunk(2 * i + 2, r_a, c_a, v_a, sem_a)

                    wait_inputs(r_b, sem_b)
                    compute_fire(base, r_b, c_b, v_b, idx_b, val_b)

                    @pl.when(i < n_pairs - 1)
                    def _():
                        load_chunk(2 * i + 3, r_b, c_b, v_b, sem_b)

                drain_streams(v_a)
                drain_streams(v_b)

                plsc.subcore_barrier()
                pltpu.sync_copy(
                    acc.at[pl.ds(s * PART, PART)],
                    w_hbm.at[pl.ds(base + s * PART, PART)])
                plsc.subcore_barrier()

    return scatter(rows_p, cols_p, vals_p, zeros)


def _mm_body(x_ref, w_ref, b_ref, o_ref):
    xb = x_ref[...]
    wb = w_ref[...].astype(jnp.bfloat16)
    acc = jnp.dot(xb, wb, preferred_element_type=jnp.float32)
    o_ref[...] = jnp.tanh(acc + b_ref[...])


def _matmul(xb, w, bias2d):
    batch = xb.shape[0]
    bn = 512
    return pl.pallas_call(
        _mm_body,
        grid=(HID // bn,),
        in_specs=[
            pl.BlockSpec((batch, IN_DIM), lambda j: (0, 0)),
            pl.BlockSpec((IN_DIM, bn), lambda j: (0, j)),
            pl.BlockSpec((1, bn), lambda j: (0, j)),
        ],
        out_specs=pl.BlockSpec((batch, bn), lambda j: (0, j)),
        out_shape=jax.ShapeDtypeStruct((batch, HID), jnp.float32),
    )(xb, w, bias2d)


def kernel(x, values, bias, rows, cols):
    nnz = rows.shape[0]
    grain = NSUB * 2 * CHUNK
    nnzp = -(-nnz // grain) * grain
    pad = nnzp - nnz
    rows_p = jnp.concatenate(
        [rows, jnp.full((pad,), IN_DIM, jnp.int32)])
    cols_p = jnp.concatenate([cols, jnp.zeros((pad,), jnp.int32)])
    vals_p = jnp.concatenate([values, jnp.zeros((pad,), jnp.float32)])
    zeros = jnp.zeros((SLICE,), jnp.float32)

    w_flat = _scatter_w(rows_p, cols_p, vals_p, zeros)
    w = w_flat[:FS].reshape(IN_DIM, HID)
    xb = x.astype(jnp.bfloat16)
    return _matmul(xb, w, bias.reshape(1, HID))


# speedup vs baseline: 11.2381x; 7.8332x over previous
import dataclasses
import functools

import jax
import jax.numpy as jnp
from jax import lax
from jax.experimental import pallas as pl
from jax.experimental.pallas import tpu as pltpu
from jax.experimental.pallas import tpu_sc as plsc

IN_DIM = 4096
HID = 4096
FS = IN_DIM * HID
NSUB = 16
NCORE = 2
SLICE = 1605632
NSLICE = -(-FS // SLICE)
PART = SLICE // NSUB
CHUNK = 2048
RROWS = 64
RING = RROWS * 128
MAXFLY = 24


def _scatter_w(rows_p, cols_p, vals_p, zeros):
    nnzp = rows_p.shape[0]
    share = nnzp // NSUB
    n_pairs = share // (2 * CHUNK)

    mesh = plsc.VectorSubcoreMesh(core_axis_name="c", subcore_axis_name="s")

    cp = pltpu.CompilerParams()
    if "needs_layout_passes" in pltpu.CompilerParams.__dataclass_fields__:
        cp = dataclasses.replace(cp, needs_layout_passes=False)

    @functools.partial(
        pl.kernel,
        compiler_params=cp,
        out_type=jax.ShapeDtypeStruct((NSLICE * SLICE,), jnp.float32),
        mesh=mesh,
        scratch_types=[
            pltpu.VMEM((CHUNK,), jnp.int32),
            pltpu.VMEM((CHUNK,), jnp.int32),
            pltpu.VMEM((CHUNK,), jnp.float32),
            pltpu.VMEM((CHUNK,), jnp.int32),
            pltpu.VMEM((CHUNK,), jnp.int32),
            pltpu.VMEM((CHUNK,), jnp.float32),
            pltpu.VMEM((RROWS, 128), jnp.int32),
            pltpu.VMEM((RROWS, 128), jnp.float32),
            pltpu.VMEM((128,), jnp.float32),
            pltpu.VMEM_SHARED((SLICE,), jnp.float32),
            pltpu.SemaphoreType.DMA,
            pltpu.SemaphoreType.DMA,
            pltpu.SemaphoreType.DMA,
        ],
    )
    def scatter(rows_hbm, cols_hbm, vals_hbm, zeros_hbm, w_hbm,
                r_a, c_a, v_a, r_b, c_b, v_b,
                ridx, rval, dr_v, acc,
                sem_a, sem_b, sem_st):
        c = lax.axis_index("c")
        s = lax.axis_index("s")
        my_off = s * share

        @pl.loop(0, RROWS)
        def _init(j):
            for l in range(8):
                ridx[j, pl.ds(l * 16, 16)] = jnp.zeros((16,), jnp.int32)

        def load_chunk(ci, r_v, c_v, v_v, sem):
            off = my_off + ci * CHUNK
            pltpu.async_copy(rows_hbm.at[pl.ds(off, CHUNK)], r_v, sem)
            pltpu.async_copy(cols_hbm.at[pl.ds(off, CHUNK)], c_v, sem)
            pltpu.async_copy(vals_hbm.at[pl.ds(off, CHUNK)], v_v, sem)

        def wait_inputs(r_v, sem):
            for _ in range(3):
                pltpu.make_async_copy(
                    rows_hbm.at[pl.ds(0, CHUNK)], r_v, sem).wait()

        def compact_chunk(base, r_v, c_v, v_v, cntv):
            def group(g, cntv):
                rv = r_v[pl.ds(g, 16)]
                cv = c_v[pl.ds(g, 16)]
                vv = v_v[pl.ds(g, 16)]
                lidx = rv * HID + cv - base
                inb = lax.bitcast_convert_type(
                    lidx, jnp.uint32) < jnp.uint32(SLICE)
                cs = plsc.cumsum(jnp.where(inb, 1, 0))
                pc = plsc.all_reduce_population_count(inb)
                dp = (cntv + cs - 1) & (RING - 1)
                row = lax.shift_right_logical(dp, 7)
                col = dp & 127
                plsc.store_scatter(ridx, [row, col], lidx, mask=inb)
                plsc.store_scatter(rval, [row, col], vv, mask=inb)
                return cntv + pc

            def body(it, cntv):
                for u in range(4):
                    cntv = group(it * 64 + u * 16, cntv)
                return cntv

            return lax.fori_loop(0, CHUNK // 64, body, cntv)

        def fire(fr):
            pltpu.async_copy(rval.at[fr], acc.at[ridx.at[fr]], sem_st,
                             add=True)

        def drain_one():
            pltpu.make_async_copy(
                zeros_hbm.at[pl.ds(0, 128)], dr_v, sem_st).wait()

        def fire_drain(cntv, fired, drained, target):
            def fire_body(st):
                f, d = st
                fire(f & (RROWS - 1))
                return f + 1, d

            fired, drained = lax.while_loop(
                lambda st: st[0] < target, fire_body, (fired, drained))

            def drain_body(st):
                f, d = st
                drain_one()
                return f, d + 1

            fired, drained = lax.while_loop(
                lambda st: st[1] + MAXFLY < st[0], drain_body,
                (fired, drained))
            return fired, drained

        @pl.loop(0, NSLICE)
        def _slice(k):
            @pl.when(k % 2 == c)
            def _():
                base = k * SLICE
                pltpu.sync_copy(zeros_hbm.at[pl.ds(s * PART, PART)],
                                acc.at[pl.ds(s * PART, PART)])
                plsc.subcore_barrier()

                load_chunk(0, r_a, c_a, v_a, sem_a)
                load_chunk(1, r_b, c_b, v_b, sem_b)

                zero16 = jnp.zeros((16,), jnp.int32)

                def pair_body(i, st):
                    cntv, fired, drained = st
                    wait_inputs(r_a, sem_a)
                    cntv = compact_chunk(base, r_a, c_a, v_a, cntv)

                    @pl.when(i < n_pairs - 1)
                    def _():
                        load_chunk(2 * i + 2, r_a, c_a, v_a, sem_a)

                    target = lax.shift_right_logical(jnp.max(cntv), 7)
                    fired, drained = fire_drain(cntv, fired, drained,
                                                target)

                    wait_inputs(r_b, sem_b)
                    cntv = compact_chunk(base, r_b, c_b, v_b, cntv)

                    @pl.when(i < n_pairs - 1)
                    def _():
                        load_chunk(2 * i + 3, r_b, c_b, v_b, sem_b)

                    target = lax.shift_right_logical(jnp.max(cntv), 7)
                    fired, drained = fire_drain(cntv, fired, drained,
                                                target)
                    return cntv, fired, drained

                cntv, fired, drained = lax.fori_loop(
                    0, n_pairs, pair_body, (zero16, 0, 0))

                cnt = jnp.max(cntv)
                lanes = jnp.arange(16, dtype=jnp.int32)
                rndup = (cnt + 127) & ~127
                for i in range(8):
                    p = cnt + i * 16 + lanes
                    m = p < rndup
                    dp = p & (RING - 1)
                    plsc.store_scatter(
                        rval,
                        [lax.shift_right_logical(dp, 7), dp & 127],
                        jnp.zeros((16,), jnp.float32), mask=m)
                target = lax.shift_right_logical(rndup, 7)
                fired, drained = fire_drain(cntv, fired, drained, target)

                def drain_rest(st):
                    f, d = st
                    drain_one()
                    return f, d + 1

                fired, drained = lax.while_loop(
                    lambda st: st[1] < st[0], drain_rest, (fired, drained))

                plsc.subcore_barrier()
                pltpu.sync_copy(
                    acc.at[pl.ds(s * PART, PART)],
                    w_hbm.at[pl.ds(base + s * PART, PART)])
                plsc.subcore_barrier()

    return scatter(rows_p, cols_p, vals_p, zeros)


def _mm_body(x_ref, w_ref, b_ref, o_ref):
    xb = x_ref[...]
    wb = w_ref[...].astype(jnp.bfloat16)
    acc = jnp.dot(xb, wb, preferred_element_type=jnp.float32)
    o_ref[...] = jnp.tanh(acc + b_ref[...])


def _matmul(xb, w, bias2d):
    batch = xb.shape[0]
    bn = 512
    return pl.pallas_call(
        _mm_body,
        grid=(HID // bn,),
        in_specs=[
            pl.BlockSpec((batch, IN_DIM), lambda j: (0, 0)),
            pl.BlockSpec((IN_DIM, bn), lambda j: (0, j)),
            pl.BlockSpec((1, bn), lambda j: (0, j)),
        ],
        out_specs=pl.BlockSpec((batch, bn), lambda j: (0, j)),
        out_shape=jax.ShapeDtypeStruct((batch, HID), jnp.float32),
    )(xb, w, bias2d)


def kernel(x, values, bias, rows, cols):
    nnz = rows.shape[0]
    grain = NSUB * 2 * CHUNK
    nnzp = -(-nnz // grain) * grain
    pad = nnzp - nnz
    rows_p = jnp.concatenate(
        [rows, jnp.full((pad,), IN_DIM, jnp.int32)])
    cols_p = jnp.concatenate([cols, jnp.zeros((pad,), jnp.int32)])
    vals_p = jnp.concatenate([values, jnp.zeros((pad,), jnp.float32)])
    zeros = jnp.zeros((SLICE,), jnp.float32)

    w_flat = _scatter_w(rows_p, cols_p, vals_p, zeros)
    w = w_flat[:FS].reshape(IN_DIM, HID)
    xb = x.astype(jnp.bfloat16)
    return _matmul(xb, w, bias.reshape(1, HID))


# speedup vs baseline: 11.5709x; 1.0296x over previous
import dataclasses
import functools

import jax
import jax.numpy as jnp
from jax import lax
from jax.experimental import pallas as pl
from jax.experimental.pallas import tpu as pltpu
from jax.experimental.pallas import tpu_sc as plsc

IN_DIM = 4096
HID = 4096
FS = IN_DIM * HID
NSUB = 16
NCORE = 2
SLICE = 1398784
NSLICE = -(-FS // SLICE)
PART = SLICE // NSUB
CHUNK = 2048
RROWS = 64
RW = 128
RWS = 7
RING = RROWS * RW
MAXFLY = 24


def _scatter_w(rows_p, cols_p, vals_p, zeros):
    nnzp = rows_p.shape[0]
    share = nnzp // NSUB
    n_pairs = share // (2 * CHUNK)

    mesh = plsc.VectorSubcoreMesh(core_axis_name="c", subcore_axis_name="s")

    cp = pltpu.CompilerParams()
    if "needs_layout_passes" in pltpu.CompilerParams.__dataclass_fields__:
        cp = dataclasses.replace(cp, needs_layout_passes=False)

    @functools.partial(
        pl.kernel,
        compiler_params=cp,
        out_type=jax.ShapeDtypeStruct((NSLICE * SLICE,), jnp.float32),
        mesh=mesh,
        scratch_types=[
            pltpu.VMEM((CHUNK,), jnp.int32),
            pltpu.VMEM((CHUNK,), jnp.int32),
            pltpu.VMEM((CHUNK,), jnp.float32),
            pltpu.VMEM((CHUNK,), jnp.int32),
            pltpu.VMEM((CHUNK,), jnp.int32),
            pltpu.VMEM((CHUNK,), jnp.float32),
            pltpu.VMEM((RROWS, RW), jnp.int32),
            pltpu.VMEM((RROWS, RW), jnp.float32),
            pltpu.VMEM((RW,), jnp.float32),
            pltpu.VMEM_SHARED((SLICE,), jnp.float32),
            pltpu.SemaphoreType.DMA,
            pltpu.SemaphoreType.DMA,
            pltpu.SemaphoreType.DMA,
        ],
    )
    def scatter(rows_hbm, cols_hbm, vals_hbm, zeros_hbm, w_hbm,
                r_a, c_a, v_a, r_b, c_b, v_b,
                ridx, rval, dr_v, acc,
                sem_a, sem_b, sem_st):
        c = lax.axis_index("c")
        s = lax.axis_index("s")
        my_off = s * share

        @pl.loop(0, RROWS)
        def _init(j):
            for l in range(RW // 16):
                ridx[j, pl.ds(l * 16, 16)] = jnp.zeros((16,), jnp.int32)

        def load_chunk(ci, r_v, c_v, v_v, sem):
            off = my_off + ci * CHUNK
            pltpu.async_copy(rows_hbm.at[pl.ds(off, CHUNK)], r_v, sem)
            pltpu.async_copy(cols_hbm.at[pl.ds(off, CHUNK)], c_v, sem)
            pltpu.async_copy(vals_hbm.at[pl.ds(off, CHUNK)], v_v, sem)

        def wait_inputs(r_v, sem):
            for _ in range(3):
                pltpu.make_async_copy(
                    rows_hbm.at[pl.ds(0, CHUNK)], r_v, sem).wait()

        def compact_chunk(base, r_v, c_v, v_v, cntv):
            def group(g, cntv):
                rv = r_v[pl.ds(g, 16)]
                cv = c_v[pl.ds(g, 16)]
                vv = v_v[pl.ds(g, 16)]
                lidx = rv * HID + cv - base
                inb = lax.bitcast_convert_type(
                    lidx, jnp.uint32) < jnp.uint32(SLICE)
                cs = plsc.cumsum(jnp.where(inb, 1, 0))
                pc = plsc.all_reduce_population_count(inb)
                dp = (cntv + cs - 1) & (RING - 1)
                row = lax.shift_right_logical(dp, RWS)
                col = dp & (RW - 1)
                plsc.store_scatter(ridx, [row, col], lidx, mask=inb)
                plsc.store_scatter(rval, [row, col], vv, mask=inb)
                return cntv + pc

            def body(it, cntv):
                for u in range(4):
                    cntv = group(it * 64 + u * 16, cntv)
                return cntv

            return lax.fori_loop(0, CHUNK // 64, body, cntv)

        def fire(fr):
            pltpu.async_copy(rval.at[fr], acc.at[ridx.at[fr]], sem_st,
                             add=True)

        def drain_one():
            pltpu.make_async_copy(
                zeros_hbm.at[pl.ds(0, RW)], dr_v, sem_st).wait()

        def fire_drain(cntv, fired, drained, target):
            def fire_body(st):
                f, d = st
                fire(f & (RROWS - 1))
                return f + 1, d

            fired, drained = lax.while_loop(
                lambda st: st[0] < target, fire_body, (fired, drained))

            def drain_body(st):
                f, d = st
                drain_one()
                return f, d + 1

            fired, drained = lax.while_loop(
                lambda st: st[1] + MAXFLY < st[0], drain_body,
                (fired, drained))
            return fired, drained

        @pl.loop(0, NSLICE)
        def _slice(k):
            @pl.when(k % 2 == c)
            def _():
                base = k * SLICE
                pltpu.sync_copy(zeros_hbm.at[pl.ds(s * PART, PART)],
                                acc.at[pl.ds(s * PART, PART)])
                plsc.subcore_barrier()

                load_chunk(0, r_a, c_a, v_a, sem_a)
                load_chunk(1, r_b, c_b, v_b, sem_b)

                zero16 = jnp.zeros((16,), jnp.int32)

                def pair_body(i, st):
                    cntv, fired, drained = st
                    wait_inputs(r_a, sem_a)
                    cntv = compact_chunk(base, r_a, c_a, v_a, cntv)

                    @pl.when(i < n_pairs - 1)
                    def _():
                        load_chunk(2 * i + 2, r_a, c_a, v_a, sem_a)

                    target = lax.shift_right_logical(jnp.max(cntv), RWS)
                    fired, drained = fire_drain(cntv, fired, drained,
                                                target)

                    wait_inputs(r_b, sem_b)
                    cntv = compact_chunk(base, r_b, c_b, v_b, cntv)

                    @pl.when(i < n_pairs - 1)
                    def _():
                        load_chunk(2 * i + 3, r_b, c_b, v_b, sem_b)

                    target = lax.shift_right_logical(jnp.max(cntv), RWS)
                    fired, drained = fire_drain(cntv, fired, drained,
                                                target)
                    return cntv, fired, drained

                cntv, fired, drained = lax.fori_loop(
                    0, n_pairs, pair_body, (zero16, 0, 0))

                cnt = jnp.max(cntv)
                lanes = jnp.arange(16, dtype=jnp.int32)
                rndup = (cnt + RW - 1) & ~(RW - 1)
                for i in range(RW // 16):
                    p = cnt + i * 16 + lanes
                    m = p < rndup
                    dp = p & (RING - 1)
                    plsc.store_scatter(
                        rval,
                        [lax.shift_right_logical(dp, RWS), dp & (RW - 1)],
                        jnp.zeros((16,), jnp.float32), mask=m)
                target = lax.shift_right_logical(rndup, RWS)
                fired, drained = fire_drain(cntv, fired, drained, target)

                def drain_rest(st):
                    f, d = st
                    drain_one()
                    return f, d + 1

                fired, drained = lax.while_loop(
                    lambda st: st[1] < st[0], drain_rest, (fired, drained))

                plsc.subcore_barrier()
                pltpu.sync_copy(
                    acc.at[pl.ds(s * PART, PART)],
                    w_hbm.at[pl.ds(base + s * PART, PART)])
                plsc.subcore_barrier()

    return scatter(rows_p, cols_p, vals_p, zeros)


def _mm_body(x_ref, w_ref, b_ref, o_ref):
    xb = x_ref[...]
    wb = w_ref[...].astype(jnp.bfloat16)
    acc = jnp.dot(xb, wb, preferred_element_type=jnp.float32)
    o_ref[...] = jnp.tanh(acc + b_ref[...])


def _matmul(xb, w, bias2d):
    batch = xb.shape[0]
    bn = 512
    return pl.pallas_call(
        _mm_body,
        grid=(HID // bn,),
        in_specs=[
            pl.BlockSpec((batch, IN_DIM), lambda j: (0, 0)),
            pl.BlockSpec((IN_DIM, bn), lambda j: (0, j)),
            pl.BlockSpec((1, bn), lambda j: (0, j)),
        ],
        out_specs=pl.BlockSpec((batch, bn), lambda j: (0, j)),
        out_shape=jax.ShapeDtypeStruct((batch, HID), jnp.float32),
    )(xb, w, bias2d)


def kernel(x, values, bias, rows, cols):
    nnz = rows.shape[0]
    grain = NSUB * 2 * CHUNK
    nnzp = -(-nnz // grain) * grain
    pad = nnzp - nnz
    rows_p = jnp.concatenate(
        [rows, jnp.full((pad,), IN_DIM, jnp.int32)])
    cols_p = jnp.concatenate([cols, jnp.zeros((pad,), jnp.int32)])
    vals_p = jnp.concatenate([values, jnp.zeros((pad,), jnp.float32)])
    zeros = jnp.zeros((SLICE,), jnp.float32)

    w_flat = _scatter_w(rows_p, cols_p, vals_p, zeros)
    w = w_flat[:FS].reshape(IN_DIM, HID)
    xb = x.astype(jnp.bfloat16)
    return _matmul(xb, w, bias.reshape(1, HID))
